# baseline (speedup 1.0000x reference)
import functools

import jax
import jax.numpy as jnp
from jax import lax
from jax.experimental import pallas as pl
from jax.experimental.pallas import tpu as pltpu
from jax.experimental.pallas import tpu_sc as plsc

EMBED_DIM = 64
CHUNK = 512
GSUB = 128


def _make_gather(B: int, D: int):
    info = plsc.get_sparse_core_info()
    NW = info.num_cores * info.num_subcores
    assert B % (NW * CHUNK) == 0
    per_w = B // NW
    n_chunks = per_w // CHUNK
    mesh = plsc.VectorSubcoreMesh(core_axis_name="c", subcore_axis_name="s")

    @functools.partial(
        pl.kernel,
        out_type=jax.ShapeDtypeStruct((B, D), jnp.float32),
        mesh=mesh,
        scratch_types=[
            pltpu.VMEM((CHUNK,), jnp.int32),
            pltpu.VMEM((CHUNK, D), jnp.float32),
            pltpu.SemaphoreType.DMA,
        ],
        compiler_params=pltpu.CompilerParams(use_tc_tiling_on_sc=False),
    )
    def gather_kernel(idx_hbm, table_hbm, out_hbm, idx_v, rows_v, sem):
        wid = lax.axis_index("s") * info.num_cores + lax.axis_index("c")
        base = wid * per_w

        def body(i, carry):
            off = base + i * CHUNK
            pltpu.sync_copy(idx_hbm.at[pl.ds(off, CHUNK)], idx_v)
            copies = [
                pltpu.async_copy(
                    table_hbm.at[idx_v.at[pl.ds(j * GSUB, GSUB)]],
                    rows_v.at[pl.ds(j * GSUB, GSUB)],
                    sem,
                )
                for j in range(CHUNK // GSUB)
            ]
            for cp in copies:
                cp.wait()
            pltpu.sync_copy(rows_v, out_hbm.at[pl.ds(off, CHUNK)])
            return carry

        lax.fori_loop(0, n_chunks, body, 0)

    return gather_kernel


def kernel(constraint_idx, table):
    B = constraint_idx.size
    D = table.shape[1]
    idx = constraint_idx.reshape(-1).astype(jnp.int32)
    out = _make_gather(B, D)(idx, table)
    return out.reshape(*constraint_idx.shape, D)


# speedup vs baseline: 1.0760x; 1.0760x over previous
import functools

import jax
import jax.numpy as jnp
from jax import lax
from jax.experimental import pallas as pl
from jax.experimental.pallas import tpu as pltpu
from jax.experimental.pallas import tpu_sc as plsc

CHUNK = 512
GSUB = 128
NBUF = 2


def _make_gather(B: int, D: int):
    info = plsc.get_sparse_core_info()
    NW = info.num_cores * info.num_subcores
    assert B % (NW * CHUNK * NBUF) == 0
    per_w = B // NW
    n_chunks = per_w // CHUNK
    mesh = plsc.VectorSubcoreMesh(core_axis_name="c", subcore_axis_name="s")

    @functools.partial(
        pl.kernel,
        out_type=jax.ShapeDtypeStruct((B, D), jnp.float32),
        mesh=mesh,
        scratch_types=[
            pltpu.VMEM((NBUF, CHUNK), jnp.int32),
            pltpu.VMEM((NBUF, CHUNK, D), jnp.float32),
            [pltpu.SemaphoreType.DMA] * NBUF,
            [pltpu.SemaphoreType.DMA] * NBUF,
            [pltpu.SemaphoreType.DMA] * NBUF,
        ],
        compiler_params=pltpu.CompilerParams(use_tc_tiling_on_sc=False),
    )
    def gather_kernel(idx_hbm, table_hbm, out_hbm, idx_v, rows_v, isem, gsem, osem):
        wid = lax.axis_index("s") * info.num_cores + lax.axis_index("c")
        base = wid * per_w

        def start_idx(c, b):
            pltpu.async_copy(
                idx_hbm.at[pl.ds(base + c * CHUNK, CHUNK)], idx_v.at[b], isem[b]
            )

        for b in range(NBUF):
            start_idx(b, b)

        def body(g, carry):
            for b in range(NBUF):
                c = g * NBUF + b

                @pl.when(g > 0)
                def _():
                    pltpu.make_async_copy(
                        rows_v.at[b], out_hbm.at[pl.ds(0, CHUNK)], osem[b]
                    ).wait()

                pltpu.make_async_copy(
                    idx_hbm.at[pl.ds(0, CHUNK)], idx_v.at[b], isem[b]
                ).wait()

                for j in range(CHUNK // GSUB):
                    pltpu.async_copy(
                        table_hbm.at[idx_v.at[b, pl.ds(j * GSUB, GSUB)]],
                        rows_v.at[b, pl.ds(j * GSUB, GSUB)],
                        gsem[b],
                    )

                for j in range(CHUNK // GSUB):
                    pltpu.make_async_copy(
                        table_hbm.at[idx_v.at[b, pl.ds(0, GSUB)]],
                        rows_v.at[b, pl.ds(0, GSUB)],
                        gsem[b],
                    ).wait()

                nc = jnp.minimum(c + NBUF, n_chunks - 1)
                start_idx(nc, b)
                pltpu.async_copy(
                    rows_v.at[b], out_hbm.at[pl.ds(base + c * CHUNK, CHUNK)], osem[b]
                )
            return carry

        lax.fori_loop(0, n_chunks // NBUF, body, 0)

        for b in range(NBUF):
            pltpu.make_async_copy(
                rows_v.at[b], out_hbm.at[pl.ds(0, CHUNK)], osem[b]
            ).wait()
            pltpu.make_async_copy(
                idx_hbm.at[pl.ds(0, CHUNK)], idx_v.at[b], isem[b]
            ).wait()

    return gather_kernel


def kernel(constraint_idx, table):
    B = constraint_idx.size
    D = table.shape[1]
    idx = constraint_idx.reshape(-1).astype(jnp.int32)
    out = _make_gather(B, D)(idx, table)
    return out.reshape(*constraint_idx.shape, D)


# speedup vs baseline: 1.0800x; 1.0038x over previous
import functools

import jax
import jax.numpy as jnp
from jax import lax
from jax.experimental import pallas as pl
from jax.experimental.pallas import tpu as pltpu
from jax.experimental.pallas import tpu_sc as plsc

CHUNK = 512
GSUB = 128
NBUF = 2


def _make_gather(B: int, D: int):
    info = plsc.get_sparse_core_info()
    NW = info.num_cores * info.num_subcores
    assert B % (NW * CHUNK * NBUF) == 0
    per_w = B // NW
    n_chunks = per_w // CHUNK
    mesh = plsc.VectorSubcoreMesh(core_axis_name="c", subcore_axis_name="s")

    @functools.partial(
        pl.kernel,
        out_type=jax.ShapeDtypeStruct((B, D), jnp.float32),
        mesh=mesh,
        scratch_types=[
            pltpu.VMEM((NBUF, CHUNK), jnp.int32),
            pltpu.VMEM((NBUF, CHUNK, D), jnp.float32),
            [pltpu.SemaphoreType.DMA] * NBUF,
            [pltpu.SemaphoreType.DMA] * NBUF,
            [pltpu.SemaphoreType.DMA] * NBUF,
        ],
        compiler_params=pltpu.CompilerParams(use_tc_tiling_on_sc=False),
    )
    def gather_kernel(idx_hbm, table_hbm, out_hbm, idx_v, rows_v, isem, gsem, osem):
        wid = lax.axis_index("s") * info.num_cores + lax.axis_index("c")
        base = wid * per_w

        def start_idx(c, b):
            pltpu.async_copy(
                idx_hbm.at[pl.ds(base + c * CHUNK, CHUNK)], idx_v.at[b], isem[b]
            )

        def wait_idx(b):
            pltpu.make_async_copy(
                idx_hbm.at[pl.ds(0, CHUNK)], idx_v.at[b], isem[b]
            ).wait()

        def fire_gathers(b):
            for j in range(CHUNK // GSUB):
                pltpu.async_copy(
                    table_hbm.at[idx_v.at[b, pl.ds(j * GSUB, GSUB)]],
                    rows_v.at[b, pl.ds(j * GSUB, GSUB)],
                    gsem[b],
                )

        def drain_gathers(b):
            for j in range(CHUNK // GSUB):
                pltpu.make_async_copy(
                    table_hbm.at[idx_v.at[b, pl.ds(0, GSUB)]],
                    rows_v.at[b, pl.ds(0, GSUB)],
                    gsem[b],
                ).wait()

        def wait_store(b):
            pltpu.make_async_copy(
                rows_v.at[b], out_hbm.at[pl.ds(0, CHUNK)], osem[b]
            ).wait()

        for b in range(NBUF):
            start_idx(b, b)
        wait_idx(0)
        fire_gathers(0)

        def body(g, carry):
            for b in range(NBUF):
                c = g * NBUF + b
                nb = (b + 1) % NBUF

                @pl.when(c < n_chunks - 1)
                def _():
                    wait_idx(nb)

                    @pl.when(c >= NBUF - 1)
                    def _():
                        wait_store(nb)

                    fire_gathers(nb)

                drain_gathers(b)
                start_idx(jnp.minimum(c + NBUF, n_chunks - 1), b)
                pltpu.async_copy(
                    rows_v.at[b], out_hbm.at[pl.ds(base + c * CHUNK, CHUNK)], osem[b]
                )
            return carry

        lax.fori_loop(0, n_chunks // NBUF, body, 0)

        for b in range(NBUF):
            wait_store(b)
            pltpu.make_async_copy(
                idx_hbm.at[pl.ds(0, CHUNK)], idx_v.at[b], isem[b]
            ).wait()

    return gather_kernel


def kernel(constraint_idx, table):
    B = constraint_idx.size
    D = table.shape[1]
    idx = constraint_idx.reshape(-1).astype(jnp.int32)
    out = _make_gather(B, D)(idx, table)
    return out.reshape(*constraint_idx.shape, D)


# speedup vs baseline: 1.7814x; 1.6494x over previous
import functools

import jax
import jax.numpy as jnp
from jax import lax
from jax.experimental import pallas as pl
from jax.experimental.pallas import tpu as pltpu
from jax.experimental.pallas import tpu_sc as plsc

CHUNK = 512
GSUB = 128
NBUF = 2


def _make_gather(B: int, D: int):
    info = plsc.get_sparse_core_info()
    NW = info.num_cores * info.num_subcores
    assert B % (NW * CHUNK * NBUF) == 0
    per_w = B // NW
    n_chunks = per_w // CHUNK
    mesh = plsc.VectorSubcoreMesh(core_axis_name="c", subcore_axis_name="s")

    @functools.partial(
        pl.kernel,
        out_type=jax.ShapeDtypeStruct((B, 2 * D), jnp.float32),
        mesh=mesh,
        scratch_types=[
            pltpu.VMEM((NBUF, CHUNK), jnp.int32),
            pltpu.VMEM((NBUF, CHUNK, D), jnp.float32),
            [pltpu.SemaphoreType.DMA] * NBUF,
            [pltpu.SemaphoreType.DMA] * NBUF,
            [pltpu.SemaphoreType.DMA] * NBUF,
        ],
        compiler_params=pltpu.CompilerParams(use_tc_tiling_on_sc=False),
    )
    def gather_kernel(idx_hbm, table_hbm, out_hbm, idx_v, rows_v, isem, gsem, osem):
        wid = lax.axis_index("s") * info.num_cores + lax.axis_index("c")
        base = wid * per_w

        def start_idx(c, b):
            pltpu.async_copy(
                idx_hbm.at[pl.ds(base + c * CHUNK, CHUNK)], idx_v.at[b], isem[b]
            )

        def wait_idx(b):
            pltpu.make_async_copy(
                idx_hbm.at[pl.ds(0, CHUNK)], idx_v.at[b], isem[b]
            ).wait()

        def fire_gathers(b):
            for j in range(CHUNK // GSUB):
                pltpu.async_copy(
                    table_hbm.at[idx_v.at[b, pl.ds(j * GSUB, GSUB)]],
                    rows_v.at[b, pl.ds(j * GSUB, GSUB)],
                    gsem[b],
                )

        def drain_gathers(b):
            for j in range(CHUNK // GSUB):
                pltpu.make_async_copy(
                    table_hbm.at[idx_v.at[b, pl.ds(0, GSUB)]],
                    rows_v.at[b, pl.ds(0, GSUB)],
                    gsem[b],
                ).wait()

        def wait_store(b):
            pltpu.make_async_copy(
                rows_v.at[b], out_hbm.at[pl.ds(0, CHUNK), pl.ds(0, D)], osem[b]
            ).wait()

        for b in range(NBUF):
            start_idx(b, b)
        wait_idx(0)
        fire_gathers(0)

        def body(g, carry):
            for b in range(NBUF):
                c = g * NBUF + b
                nb = (b + 1) % NBUF

                @pl.when(c < n_chunks - 1)
                def _():
                    wait_idx(nb)

                    @pl.when(c >= NBUF - 1)
                    def _():
                        wait_store(nb)

                    fire_gathers(nb)

                drain_gathers(b)
                start_idx(jnp.minimum(c + NBUF, n_chunks - 1), b)
                pltpu.async_copy(
                    rows_v.at[b],
                    out_hbm.at[pl.ds(base + c * CHUNK, CHUNK), pl.ds(0, D)],
                    osem[b],
                )
            return carry

        lax.fori_loop(0, n_chunks // NBUF, body, 0)

        for b in range(NBUF):
            wait_store(b)
            pltpu.make_async_copy(
                idx_hbm.at[pl.ds(0, CHUNK)], idx_v.at[b], isem[b]
            ).wait()

    return gather_kernel


def kernel(constraint_idx, table):
    B = constraint_idx.size
    D = table.shape[1]
    idx = constraint_idx.reshape(-1).astype(jnp.int32)
    out = _make_gather(B, D)(idx, table)
    return out[:, :D].reshape(*constraint_idx.shape, D)
